# baseline (speedup 1.0000x reference)
import jax
import jax.numpy as jnp
from jax import lax
from jax.experimental import pallas as pl
from jax.experimental.pallas import tpu as pltpu
from jax.experimental.pallas import tpu_sc as plsc

_B = 16
_N0 = 2048
_NE = 2048
_K = 32
_L = 16
_NW = 32


def _sc_body(pts_x, pts_y, ext, csx, csy, rv, out,
             vx, vy, ve, ccx, ccy, ccy2, rvv, accm, outv, sem):
    del sem
    wid = lax.axis_index("s") * 2 + lax.axis_index("c")

    pltpu.sync_copy(pts_x.at[wid], vx)
    pltpu.sync_copy(pts_y.at[wid], vy)
    pltpu.sync_copy(ext.at[wid], ve)
    pltpu.sync_copy(csx, ccx)
    pltpu.sync_copy(csy, ccy)
    pltpu.sync_copy(rv, rvv)

    rr = jnp.abs(rvv[...])
    zeros = jnp.zeros((_L,), jnp.float32)
    for k in range(_K):
        accm[pl.ds(k * _L, _L)] = zeros
        ccy2[k] = 1.0 - ccy[k]

    def main_body(j, carry):
        base = pl.multiple_of(j * _L, _L)
        px = vx[pl.ds(base, _L)]
        py = vy[pl.ds(base, _L)]
        for k in range(_K):
            d = jnp.abs(px - ccx[k]) + jnp.abs(py - ccy[k])
            w = jnp.abs(rr - d)
            a = pl.ds(k * _L, _L)
            accm[a] = accm[a] + (w - d) / ((1.0 + d) * (1.0 + w))
        return carry

    lax.fori_loop(0, _N0 // _L, main_body, 0)

    def ext_body(j, carry):
        base = pl.multiple_of(j * _L, _L)
        t = ve[pl.ds(base, _L)]
        for k in range(_K):
            d = jnp.abs(t - ccx[k]) + jnp.abs(t - ccy2[k])
            w = jnp.abs(rr - d)
            a = pl.ds(k * _L, _L)
            accm[a] = accm[a] + (w - d) / ((1.0 + d) * (1.0 + w))
        return carry

    lax.fori_loop(0, _NE // _L, ext_body, 0)

    lanes = lax.iota(jnp.int32, _L)
    dn = lax.GatherDimensionNumbers(
        offset_dims=(), collapsed_slice_dims=(0,), start_index_map=(0,))
    perms = [(lanes ^ sh)[:, None] for sh in (8, 4, 2, 1)]

    def _permute(a, idx):
        return lax.gather(a, idx, dn, slice_sizes=(1,),
                          mode=lax.GatherScatterMode.PROMISE_IN_BOUNDS)

    for g in range(_K // _L):
        s = jnp.zeros((_L,), jnp.float32)
        for c in range(_L):
            a = accm[pl.ds((g * _L + c) * _L, _L)]
            for idx in perms:
                a = a + _permute(a, idx)
            s = jnp.where(lanes == c, a, s)
        outv[pl.ds(g * _L, _L)] = s

    pltpu.sync_copy(outv, out.at[wid])


def _tc_body(xo_ref, x_ref, tpl_ref):
    up = xo_ref[0:_B, :]
    dn = xo_ref[_B:2 * _B, :]
    x_ref[...] = jnp.concatenate([up, dn], axis=1)
    diff = up - dn
    tpl_ref[...] = (-jnp.sum(diff * diff))[None, None]


def kernel(beta_0_up, beta_0_down, beta0_ext, beta1_ext, centers, radius):
    pts_x = jnp.concatenate([beta_0_up[:, :, 0], beta_0_down[:, :, 0]], axis=0)
    pts_y = jnp.concatenate([beta_0_up[:, :, 1], beta_0_down[:, :, 1]], axis=0)
    ext_t = jnp.concatenate([
        jnp.concatenate([beta0_ext[:, :, 1], beta1_ext[:, :, 1]], axis=1),
        jnp.concatenate([beta0_ext[:, :, 0], beta1_ext[:, :, 0]], axis=1),
    ], axis=0)
    csx = jnp.broadcast_to(centers[:, 0:1], (_K, _L))
    csy = jnp.broadcast_to(centers[:, 1:2], (_K, _L))
    rv = jnp.broadcast_to(radius, (_L,))

    mesh = plsc.VectorSubcoreMesh(core_axis_name="c", subcore_axis_name="s")
    xo = pl.kernel(
        _sc_body,
        out_type=jax.ShapeDtypeStruct((_NW, _K), jnp.float32),
        mesh=mesh,
        scratch_types=[
            pltpu.VMEM((_N0,), jnp.float32),
            pltpu.VMEM((_N0,), jnp.float32),
            pltpu.VMEM((_NE,), jnp.float32),
            pltpu.VMEM((_K, _L), jnp.float32),
            pltpu.VMEM((_K, _L), jnp.float32),
            pltpu.VMEM((_K, _L), jnp.float32),
            pltpu.VMEM((_L,), jnp.float32),
            pltpu.VMEM((_K * _L,), jnp.float32),
            pltpu.VMEM((_K,), jnp.float32),
            pltpu.SemaphoreType.DMA,
        ],
    )(pts_x, pts_y, ext_t, csx, csy, rv)

    x, tpl = pl.pallas_call(
        _tc_body,
        out_shape=(
            jax.ShapeDtypeStruct((_B, 2 * _K), jnp.float32),
            jax.ShapeDtypeStruct((1, 1), jnp.float32),
        ),
    )(xo)
    return (x, tpl[0, 0])


# speedup vs baseline: 1.0626x; 1.0626x over previous
import jax
import jax.numpy as jnp
from jax import lax
from jax.experimental import pallas as pl
from jax.experimental.pallas import tpu as pltpu
from jax.experimental.pallas import tpu_sc as plsc

_B = 16
_N0 = 2048
_NE = 2048
_K = 32
_L = 16
_NW = 32


def _sc_body(pts_x, pts_y, ext, csx, csy, rv, out,
             vx, vy, ve, ccx, ccy, ccy2, rvv, accm, outv, sem):
    del sem
    wid = lax.axis_index("s") * 2 + lax.axis_index("c")

    pltpu.sync_copy(pts_x.at[wid], vx)
    pltpu.sync_copy(pts_y.at[wid], vy)
    pltpu.sync_copy(ext.at[wid], ve)
    pltpu.sync_copy(csx, ccx)
    pltpu.sync_copy(csy, ccy)
    pltpu.sync_copy(rv, rvv)

    rr = jnp.abs(rvv[...])
    zeros = jnp.zeros((_L,), jnp.float32)
    del ccy2

    G = 8
    for g0 in range(0, _K, G):
        cxs = [ccx[k] for k in range(g0, g0 + G)]
        cys = [ccy[k] for k in range(g0, g0 + G)]
        cy2s = [1.0 - c for c in cys]

        def main_body(j, accs, _cxs=cxs, _cys=cys):
            base = pl.multiple_of(j * _L, _L)
            px = vx[pl.ds(base, _L)]
            py = vy[pl.ds(base, _L)]
            out = []
            for i in range(G):
                d = jnp.abs(px - _cxs[i]) + jnp.abs(py - _cys[i])
                w = jnp.abs(rr - d)
                out.append(accs[i] + (w - d) / ((1.0 + d) * (1.0 + w)))
            return tuple(out)

        def ext_body(j, accs, _cxs=cxs, _cy2s=cy2s):
            base = pl.multiple_of(j * _L, _L)
            t = ve[pl.ds(base, _L)]
            out = []
            for i in range(G):
                d = jnp.abs(t - _cxs[i]) + jnp.abs(t - _cy2s[i])
                w = jnp.abs(rr - d)
                out.append(accs[i] + (w - d) / ((1.0 + d) * (1.0 + w)))
            return tuple(out)

        accs = lax.fori_loop(0, _N0 // _L, main_body, (zeros,) * G)
        accs = lax.fori_loop(0, _NE // _L, ext_body, accs)
        for i in range(G):
            accm[pl.ds((g0 + i) * _L, _L)] = accs[i]

    lanes = lax.iota(jnp.int32, _L)
    dn = lax.GatherDimensionNumbers(
        offset_dims=(), collapsed_slice_dims=(0,), start_index_map=(0,))
    perms = [(lanes ^ sh)[:, None] for sh in (8, 4, 2, 1)]

    def _permute(a, idx):
        return lax.gather(a, idx, dn, slice_sizes=(1,),
                          mode=lax.GatherScatterMode.PROMISE_IN_BOUNDS)

    for g in range(_K // _L):
        s = jnp.zeros((_L,), jnp.float32)
        for c in range(_L):
            a = accm[pl.ds((g * _L + c) * _L, _L)]
            for idx in perms:
                a = a + _permute(a, idx)
            s = jnp.where(lanes == c, a, s)
        outv[pl.ds(g * _L, _L)] = s

    pltpu.sync_copy(outv, out.at[wid])


def _tc_body(xo_ref, x_ref, tpl_ref):
    up = xo_ref[0:_B, :]
    dn = xo_ref[_B:2 * _B, :]
    x_ref[...] = jnp.concatenate([up, dn], axis=1)
    diff = up - dn
    tpl_ref[...] = (-jnp.sum(diff * diff))[None, None]


def kernel(beta_0_up, beta_0_down, beta0_ext, beta1_ext, centers, radius):
    pts_x = jnp.concatenate([beta_0_up[:, :, 0], beta_0_down[:, :, 0]], axis=0)
    pts_y = jnp.concatenate([beta_0_up[:, :, 1], beta_0_down[:, :, 1]], axis=0)
    ext_t = jnp.concatenate([
        jnp.concatenate([beta0_ext[:, :, 1], beta1_ext[:, :, 1]], axis=1),
        jnp.concatenate([beta0_ext[:, :, 0], beta1_ext[:, :, 0]], axis=1),
    ], axis=0)
    csx = jnp.broadcast_to(centers[:, 0:1], (_K, _L))
    csy = jnp.broadcast_to(centers[:, 1:2], (_K, _L))
    rv = jnp.broadcast_to(radius, (_L,))

    mesh = plsc.VectorSubcoreMesh(core_axis_name="c", subcore_axis_name="s")
    xo = pl.kernel(
        _sc_body,
        out_type=jax.ShapeDtypeStruct((_NW, _K), jnp.float32),
        mesh=mesh,
        scratch_types=[
            pltpu.VMEM((_N0,), jnp.float32),
            pltpu.VMEM((_N0,), jnp.float32),
            pltpu.VMEM((_NE,), jnp.float32),
            pltpu.VMEM((_K, _L), jnp.float32),
            pltpu.VMEM((_K, _L), jnp.float32),
            pltpu.VMEM((_K, _L), jnp.float32),
            pltpu.VMEM((_L,), jnp.float32),
            pltpu.VMEM((_K * _L,), jnp.float32),
            pltpu.VMEM((_K,), jnp.float32),
            pltpu.SemaphoreType.DMA,
        ],
    )(pts_x, pts_y, ext_t, csx, csy, rv)

    x, tpl = pl.pallas_call(
        _tc_body,
        out_shape=(
            jax.ShapeDtypeStruct((_B, 2 * _K), jnp.float32),
            jax.ShapeDtypeStruct((1, 1), jnp.float32),
        ),
    )(xo)
    return (x, tpl[0, 0])
